# Initial kernel scaffold; baseline (speedup 1.0000x reference)
import jax
import jax.numpy as jnp
from jax.experimental import pallas as pl


def kernel(x, edge_index_rel0, edge_index_rel1, Ws1_0, Wn1_0, b1_0, Ws1_1, Wn1_1, b1_1, Ws2_0, Wn2_0, b2_0, Ws2_1, Wn2_1, b2_1):
    raise NotImplementedError("write your pallas kernel here")


# speedup vs baseline: 3.7648x; 3.7648x over previous
import functools

import jax
import jax.numpy as jnp
from jax import lax
from jax.experimental import pallas as pl
from jax.experimental.pallas import tpu as pltpu
from jax.experimental.pallas import tpu_sc as plsc

N = 10000
E = 160000
IN, HID, OUT = 256, 512, 256
H = IN // 2

NS = 16
EPT = E // NS
CH = 128
NCH = -(-EPT // CH)
EPAD = NCH * CH
RPT = 632
ACC_N = NS * RPT
LAST = NS - 1
RLAST = N - LAST * RPT

_mesh = plsc.VectorSubcoreMesh(core_axis_name="c", subcore_axis_name="s")


@functools.partial(
    pl.kernel,
    out_type=jax.ShapeDtypeStruct((2 * N, H), jnp.float32),
    mesh=_mesh,
    scratch_types=[
        pltpu.VMEM_SHARED((ACC_N, H), jnp.float32),
        pltpu.VMEM((1, CH), jnp.int32),
        pltpu.VMEM((1, CH), jnp.int32),
        pltpu.VMEM((CH, H), jnp.float32),
        pltpu.SemaphoreType.DMA,
    ],
)
def _seg_sum(tLR, srcp, dstp, z128, agg2,
             acc, src_v, dst_v, rows_v, sem):
    c = lax.axis_index("c")
    s = lax.axis_index("s")
    r0 = s * RPT
    bias = c * N

    pltpu.sync_copy(z128, acc.at[pl.ds(r0, RPT)])
    plsc.subcore_barrier()

    def chunk(j, carry):
        pltpu.sync_copy(srcp.at[s * NCH + j], src_v)
        pltpu.sync_copy(dstp.at[s * NCH + j], dst_v)
        for q in range(CH // 16):
            sl = pl.ds(q * 16, 16)
            src_v[0, sl] = src_v[0, sl] + bias
        pltpu.async_copy(tLR.at[src_v.at[0]], rows_v, sem).wait()
        pltpu.sync_copy(rows_v, acc.at[dst_v.at[0]], add=True)
        return carry

    lax.fori_loop(0, NCH, chunk, 0)
    plsc.subcore_barrier()

    ob = bias + r0

    @pl.when(s < LAST)
    def _():
        pltpu.sync_copy(acc.at[pl.ds(r0, RPT)], agg2.at[pl.ds(ob, RPT)])

    @pl.when(s == LAST)
    def _():
        pltpu.sync_copy(acc.at[pl.ds(r0, RLAST)], agg2.at[pl.ds(ob, RLAST)])


@functools.partial(
    pl.kernel,
    out_type=jax.ShapeDtypeStruct((2 * N, H), jnp.float32),
    mesh=_mesh,
    scratch_types=[
        pltpu.VMEM_SHARED((ACC_N, H), jnp.float32),
        pltpu.VMEM((1, CH), jnp.int32),
        pltpu.VMEM((CH, H), jnp.float32),
    ],
)
def _deg_both(dstp2, z128, ones128, deg2,
              degacc, dst_v, ones_v):
    c = lax.axis_index("c")
    s = lax.axis_index("s")
    r0 = s * RPT

    pltpu.sync_copy(z128, degacc.at[pl.ds(r0, RPT)])
    pltpu.sync_copy(ones128, ones_v)
    plsc.subcore_barrier()

    def chunk(j, carry):
        pltpu.sync_copy(dstp2.at[(c * NS + s) * NCH + j], dst_v)
        pltpu.sync_copy(ones_v, degacc.at[dst_v.at[0]], add=True)
        return carry

    lax.fori_loop(0, NCH, chunk, 0)
    plsc.subcore_barrier()

    ob = c * N + r0

    @pl.when(s < LAST)
    def _():
        pltpu.sync_copy(degacc.at[pl.ds(r0, RPT)], deg2.at[pl.ds(ob, RPT)])

    @pl.when(s == LAST)
    def _():
        pltpu.sync_copy(degacc.at[pl.ds(r0, RLAST)], deg2.at[pl.ds(ob, RLAST)])


BM = 1000
NB = N // BM


def _tc_main_body(x, a0L, a0R, a1L, a1R, d0, d1, W1s, W1n0, W1n1, b1, W2, b2,
                  hs, y0L, y0R, y1L, y1R):
    i0 = 1.0 / jnp.maximum(d0[:, :1], 1.0)
    i1 = 1.0 / jnp.maximum(d1[:, :1], 1.0)
    a0 = jnp.concatenate([a0L[...], a0R[...]], axis=1) * i0
    a1 = jnp.concatenate([a1L[...], a1R[...]], axis=1) * i1
    h = (jnp.dot(x[...], W1s[...], preferred_element_type=jnp.float32)
         + jnp.dot(a0, W1n0[...], preferred_element_type=jnp.float32)
         + jnp.dot(a1, W1n1[...], preferred_element_type=jnp.float32)
         + b1[...])
    h = jnp.maximum(h, 0.0)
    t2 = jnp.dot(h, W2[...], preferred_element_type=jnp.float32)
    hs[...] = t2[:, :OUT] + b2[...]
    y0L[...] = t2[:, OUT:OUT + H]
    y0R[...] = t2[:, OUT + H:2 * OUT]
    y1L[...] = t2[:, 2 * OUT:2 * OUT + H]
    y1R[...] = t2[:, 2 * OUT + H:]


def _tc_final_body(hs, a0L, a0R, a1L, a1R, d0, d1, out):
    i0 = 1.0 / jnp.maximum(d0[:, :1], 1.0)
    i1 = 1.0 / jnp.maximum(d1[:, :1], 1.0)
    add = jnp.concatenate(
        [a0L[...] * i0 + a1L[...] * i1, a0R[...] * i0 + a1R[...] * i1], axis=1)
    out[...] = jnp.maximum(hs[...] + add, 0.0)


def _rowL(w):
    return pl.BlockSpec((BM, w), lambda i: (i, 0))


def _rowR(w):
    return pl.BlockSpec((BM, w), lambda i: (i + NB, 0))


def _full_spec(shape):
    return pl.BlockSpec(shape, lambda i: (0,) * len(shape))


def _prep_edges(edge_index):
    src = edge_index[0].reshape(NS, EPT)
    dst = edge_index[1].reshape(NS, EPT)
    src = jnp.pad(src, ((0, 0), (0, EPAD - EPT)))
    dst = jnp.pad(dst, ((0, 0), (0, EPAD - EPT)), constant_values=N)
    return src.reshape(NS * NCH, 1, CH), dst.reshape(NS * NCH, 1, CH)


def kernel(x, edge_index_rel0, edge_index_rel1,
           Ws1_0, Wn1_0, b1_0, Ws1_1, Wn1_1, b1_1,
           Ws2_0, Wn2_0, b2_0, Ws2_1, Wn2_1, b2_1):
    src0, dst0 = _prep_edges(edge_index_rel0)
    src1, dst1 = _prep_edges(edge_index_rel1)
    xLR = jnp.concatenate([x[:, :H], x[:, H:]], axis=0)
    z128 = jnp.zeros((RPT, H), jnp.float32)
    ones128 = jnp.ones((CH, H), jnp.float32)

    deg2 = _deg_both(jnp.concatenate([dst0, dst1], axis=0), z128, ones128)
    agg0 = _seg_sum(xLR, src0, dst0, z128)
    agg1 = _seg_sum(xLR, src1, dst1, z128)

    W1s = Ws1_0 + Ws1_1
    b1 = (b1_0 + b1_1).reshape(1, HID)
    W2 = jnp.concatenate([Ws2_0 + Ws2_1, Wn2_0, Wn2_1], axis=1)
    b2 = (b2_0 + b2_1).reshape(1, OUT)

    grid = (NB,)
    hs, y0L, y0R, y1L, y1R = pl.pallas_call(
        _tc_main_body,
        grid=grid,
        in_specs=[
            _rowL(IN), _rowL(H), _rowR(H), _rowL(H), _rowR(H),
            _rowL(H), _rowR(H),
            _full_spec((IN, HID)), _full_spec((IN, HID)), _full_spec((IN, HID)),
            _full_spec((1, HID)), _full_spec((HID, 3 * OUT)),
            _full_spec((1, OUT)),
        ],
        out_specs=[_rowL(OUT), _rowL(H), _rowL(H), _rowL(H), _rowL(H)],
        out_shape=[
            jax.ShapeDtypeStruct((N, OUT), jnp.float32),
            jax.ShapeDtypeStruct((N, H), jnp.float32),
            jax.ShapeDtypeStruct((N, H), jnp.float32),
            jax.ShapeDtypeStruct((N, H), jnp.float32),
            jax.ShapeDtypeStruct((N, H), jnp.float32),
        ],
    )(x, agg0, agg0, agg1, agg1, deg2, deg2, W1s, Wn1_0, Wn1_1, b1, W2, b2)

    y0 = jnp.concatenate([y0L, y0R], axis=0)
    y1 = jnp.concatenate([y1L, y1R], axis=0)
    agg20 = _seg_sum(y0, src0, dst0, z128)
    agg21 = _seg_sum(y1, src1, dst1, z128)

    out = pl.pallas_call(
        _tc_final_body,
        grid=grid,
        in_specs=[_rowL(OUT), _rowL(H), _rowR(H), _rowL(H), _rowR(H),
                  _rowL(H), _rowR(H)],
        out_specs=_rowL(OUT),
        out_shape=jax.ShapeDtypeStruct((N, OUT), jnp.float32),
    )(hs, agg20, agg20, agg21, agg21, deg2, deg2)
    return out
